# baseline (speedup 1.0000x reference)
import functools

import jax
import jax.numpy as jnp
from jax import lax
from jax.experimental import pallas as pl
from jax.experimental.pallas import tpu as pltpu, tpu_sc as plsc

N = 10000
H = 256
NEG_SLOPE = 0.01
EPS = 1e-5

NC, NS = 2, 16
NW = NC * NS
RANGE = 320
NPAD = NW * RANGE
NINF = float("-inf")

SEGBLK = 512
MSLOT = 16
_SC_MESH = dict(core_axis_name="c", subcore_axis_name="s")


def _wid():
    return lax.axis_index("s") * NC + lax.axis_index("c")


def _bucket(dst):
    return jnp.minimum((dst * 13108) >> 22, 31)


def _make_bucketize(ept, arn):
    nblk = ept // 1024

    @functools.partial(
        pl.kernel,
        out_type=[
            jax.ShapeDtypeStruct((NW * arn,), jnp.int32),
            jax.ShapeDtypeStruct((NW * 2 * NW * MSLOT,), jnp.int32),
        ],
        mesh=plsc.VectorSubcoreMesh(**_SC_MESH),
        scratch_types=[
            pltpu.VMEM((arn,), jnp.int32),
            pltpu.VMEM((1024,), jnp.int32),
            pltpu.VMEM((1024,), jnp.int32),
            pltpu.VMEM((16,), jnp.int32),
            pltpu.SMEM((64,), jnp.int32),
        ],
    )
    def bucketize(src_hbm, dst_hbm, arena_o, meta_o, arena, dstst, srcst, mbuf, cnt):
        w = _wid()
        iota = lax.iota(jnp.int32, 16)
        ebase = w * ept

        for b in range(NW):
            cnt[b] = 0

        def p1_blk(blk, _):
            pltpu.sync_copy(
                dst_hbm.at[pl.ds(pl.multiple_of(ebase + blk * 1024, 16), 1024)], dstst)

            def p1_chunk(ci, _):
                bv = _bucket(dstst[pl.ds(ci * 16, 16)])
                for e in range(16):
                    b = bv[e]
                    cnt[b] = cnt[b] + 1
                return 0

            lax.fori_loop(0, 64, p1_chunk, 0)
            return 0

        lax.fori_loop(0, nblk, p1_blk, 0)

        run = jnp.int32(0)
        for b in range(NW):
            c = cnt[b]
            cnt[b] = run
            cnt[32 + b] = run
            run = run + ((c + 15) & ~15)

        dummy = jnp.full((16,), RANGE * 16384, jnp.int32)

        def fill(i, _):
            arena[pl.ds(i * 16, 16)] = dummy
            return 0

        lax.fori_loop(0, arn // 16, fill, 0)

        def p2_blk(blk, _):
            base = pl.multiple_of(ebase + blk * 1024, 16)
            pltpu.sync_copy(dst_hbm.at[pl.ds(base, 1024)], dstst)
            pltpu.sync_copy(src_hbm.at[pl.ds(base, 1024)], srcst)

            def p2_chunk(ci, _):
                dv = dstst[pl.ds(ci * 16, 16)]
                sv = srcst[pl.ds(ci * 16, 16)]
                bv = _bucket(dv)
                pkv = (dv - bv * RANGE) * 16384 + sv
                for e in range(16):
                    b = bv[e]
                    pk = pkv[e]
                    p = cnt[32 + b]
                    base = p & ~15
                    old = arena[pl.ds(base, 16)]
                    arena[pl.ds(base, 16)] = jnp.where(iota == (p & 15), pk, old)
                    cnt[32 + b] = p + 1
                return 0

            lax.fori_loop(0, 64, p2_chunk, 0)
            return 0

        lax.fori_loop(0, nblk, p2_blk, 0)

        for b in range(NW):
            c16 = (cnt[32 + b] - cnt[b] + 15) & ~15
            mbuf[...] = jnp.where(iota == 0, c16, 0)
            pltpu.sync_copy(mbuf.at[pl.ds(0, MSLOT)],
                            meta_o.at[pl.ds(pl.multiple_of((b * 2) * NW * MSLOT + w * MSLOT, 16), MSLOT)])
            mbuf[...] = jnp.where(iota == 0, cnt[b], 0)
            pltpu.sync_copy(mbuf.at[pl.ds(0, MSLOT)],
                            meta_o.at[pl.ds(pl.multiple_of((b * 2 + 1) * NW * MSLOT + w * MSLOT, 16), MSLOT)])

        pltpu.sync_copy(arena.at[pl.ds(0, arn)],
                        arena_o.at[pl.ds(pl.multiple_of(w * arn, 16), arn)])

    return bucketize


def _make_segmax(arn, d):
    nvr = d // 16

    @functools.partial(
        pl.kernel,
        out_type=jax.ShapeDtypeStruct((NPAD * d,), jnp.float32),
        mesh=plsc.VectorSubcoreMesh(**_SC_MESH),
        scratch_types=[
            pltpu.VMEM(((RANGE + 1) * d,), jnp.float32),
            pltpu.VMEM((SEGBLK + 16,), jnp.int32),
            pltpu.VMEM((16, d), jnp.float32),
            pltpu.VMEM((16, d), jnp.float32),
            pltpu.VMEM((2 * NW * MSLOT,), jnp.int32),
            pltpu.SemaphoreType.DMA,
            pltpu.SemaphoreType.DMA,
        ],
    )
    def segmax(h_hbm, arena_hbm, meta_hbm, out_hbm, acc, seg, rows_a, rows_b,
               meta_v, sem_a, sem_b):
        w = _wid()
        ninf = jnp.full((16,), NINF, jnp.float32)
        dummy = jnp.full((16,), RANGE * 16384, jnp.int32)

        def init(i, _):
            acc[pl.ds(i * 16, 16)] = ninf
            return 0

        lax.fori_loop(0, (RANGE + 1) * d // 16, init, 0)

        pltpu.sync_copy(
            meta_hbm.at[pl.ds(pl.multiple_of(w * 2 * NW * MSLOT, 16), 2 * NW * MSLOT)],
            meta_v)

        def accum(rows, dstv):
            for e in range(16):
                base = dstv[e] * d
                for j in range(nvr):
                    a = acc[pl.ds(base + j * 16, 16)]
                    r = rows[e, pl.ds(j * 16, 16)]
                    acc[pl.ds(base + j * 16, 16)] = jnp.maximum(a, r)

        def producer(t, _):
            cntp = meta_v[pl.ds(t * MSLOT, 16)][0]
            off = meta_v[pl.ds(NW * MSLOT + t * MSLOT, 16)][0]
            abase = pl.multiple_of(t * arn + off, 16)

            def blk(bi, _):
                pltpu.sync_copy(
                    arena_hbm.at[pl.ds(pl.multiple_of(abase + bi * SEGBLK, 16), SEGBLK)],
                    seg.at[pl.ds(0, SEGBLK)])
                nch = jnp.minimum(cntp - bi * SEGBLK, SEGBLK) >> 4
                seg[pl.ds(nch * 16, 16)] = dummy
                npair = (nch + 1) >> 1

                pltpu.async_copy(h_hbm.at[seg[pl.ds(0, 16)] & 16383], rows_a, sem_a)

                def pair(pi, _):
                    ca = pi * 2
                    pva = seg[pl.ds(ca * 16, 16)]
                    dsta = lax.shift_right_logical(pva, 14)
                    pvb = seg[pl.ds(ca * 16 + 16, 16)]
                    ivb = pvb & 16383
                    dstb = lax.shift_right_logical(pvb, 14)
                    pltpu.async_copy(h_hbm.at[ivb], rows_b, sem_b)
                    pltpu.make_async_copy(h_hbm.at[pl.ds(0, 16)], rows_a, sem_a).wait()
                    accum(rows_a, dsta)
                    nxt = jnp.minimum(ca + 2, nch)
                    ivn = seg[pl.ds(nxt * 16, 16)] & 16383
                    pltpu.async_copy(h_hbm.at[ivn], rows_a, sem_a)
                    pltpu.make_async_copy(h_hbm.at[pl.ds(0, 16)], rows_b, sem_b).wait()
                    accum(rows_b, dstb)
                    return 0

                lax.fori_loop(0, npair, pair, 0)
                pltpu.make_async_copy(h_hbm.at[pl.ds(0, 16)], rows_a, sem_a).wait()
                return 0

            lax.fori_loop(0, (cntp + SEGBLK - 1) >> 9, blk, 0)
            return 0

        lax.fori_loop(0, NW, producer, 0)

        def fin(i, _):
            a = acc[pl.ds(i * 16, 16)]
            acc[pl.ds(i * 16, 16)] = jnp.where(a == NINF, 0.0, a)
            return 0

        lax.fori_loop(0, RANGE * d // 16, fin, 0)
        pltpu.sync_copy(acc.at[pl.ds(0, RANGE * d)],
                        out_hbm.at[pl.ds(pl.multiple_of(w * RANGE * d, 16), RANGE * d)])

    return segmax


_GRID = NPAD // 256


def _tc_linear(agg, h, wl, wr):
    k = agg.shape[1]

    def body(agg_ref, h_ref, wl_ref, wr_ref, y_ref, st_ref, sacc):
        i = pl.program_id(0)
        y = lax.dot_general(agg_ref[...], wl_ref[...], (((1,), (1,)), ((), ())),
                            preferred_element_type=jnp.float32)
        y = y + lax.dot_general(h_ref[...], wr_ref[...], (((1,), (1,)), ((), ())),
                                preferred_element_type=jnp.float32)
        y_ref[...] = y
        rid = i * 256 + lax.broadcasted_iota(jnp.int32, (256, 1), 0)
        ym = jnp.where(rid < N, y, 0.0)

        @pl.when(i == 0)
        def _():
            sacc[...] = jnp.zeros((8, H), jnp.float32)

        sacc[0:1, :] = sacc[0:1, :] + jnp.sum(ym, axis=0, keepdims=True)
        sacc[1:2, :] = sacc[1:2, :] + jnp.sum(ym * ym, axis=0, keepdims=True)

        @pl.when(i == _GRID - 1)
        def _():
            st_ref[...] = sacc[...]

    return pl.pallas_call(
        body,
        grid=(_GRID,),
        in_specs=[
            pl.BlockSpec((256, k), lambda i: (i, 0)),
            pl.BlockSpec((256, k), lambda i: (i, 0)),
            pl.BlockSpec((H, k), lambda i: (0, 0)),
            pl.BlockSpec((H, k), lambda i: (0, 0)),
        ],
        out_specs=[
            pl.BlockSpec((256, H), lambda i: (i, 0)),
            pl.BlockSpec((8, H), lambda i: (0, 0)),
        ],
        out_shape=[
            jax.ShapeDtypeStruct((NPAD, H), jnp.float32),
            jax.ShapeDtypeStruct((8, H), jnp.float32),
        ],
        scratch_shapes=[pltpu.VMEM((8, H), jnp.float32)],
    )(agg, h, wl, wr)


def _tc_bn(y, st, gb):
    def body(y_ref, st_ref, gb_ref, o_ref):
        s = st_ref[0:1, :]
        s2 = st_ref[1:2, :]
        mean = s / N
        var = s2 / N - mean * mean
        rstd = lax.rsqrt(var + EPS)
        g = gb_ref[0:1, :]
        b = gb_ref[1:2, :]
        hn = (y_ref[...] - mean) * (rstd * g) + b
        o_ref[...] = jnp.where(hn >= 0, hn, hn * NEG_SLOPE)

    return pl.pallas_call(
        body,
        grid=(_GRID,),
        in_specs=[
            pl.BlockSpec((256, H), lambda i: (i, 0)),
            pl.BlockSpec((8, H), lambda i: (0, 0)),
            pl.BlockSpec((8, H), lambda i: (0, 0)),
        ],
        out_specs=pl.BlockSpec((256, H), lambda i: (i, 0)),
        out_shape=jax.ShapeDtypeStruct((NPAD, H), jnp.float32),
    )(y, st, gb)


def _tc_head(h, fc1w, fc1b, fc2w, fc2b):
    def body(h_ref, w1_ref, b1_ref, w2_ref, b2_ref, o_ref):
        h2 = lax.dot_general(h_ref[...], w1_ref[...], (((1,), (1,)), ((), ())),
                             preferred_element_type=jnp.float32)
        h2 = h2 + b1_ref[0:1, :]
        h2 = jnp.where(h2 >= 0, h2, h2 * NEG_SLOPE)
        o = jnp.sum(h2 * w2_ref[0:1, :], axis=1, keepdims=True) + b2_ref[0, 0]
        o_ref[...] = jnp.broadcast_to(o, (256, 128))

    return pl.pallas_call(
        body,
        grid=(_GRID,),
        in_specs=[
            pl.BlockSpec((256, H), lambda i: (i, 0)),
            pl.BlockSpec((128, H), lambda i: (0, 0)),
            pl.BlockSpec((8, 128), lambda i: (0, 0)),
            pl.BlockSpec((8, 128), lambda i: (0, 0)),
            pl.BlockSpec((8, 128), lambda i: (0, 0)),
        ],
        out_specs=pl.BlockSpec((256, 128), lambda i: (i, 0)),
        out_shape=jax.ShapeDtypeStruct((NPAD, 128), jnp.float32),
    )(h, fc1w, fc1b, fc2w, fc2b)


def _row8(v, width):
    return jnp.broadcast_to(v.reshape(1, -1), (8, width)).astype(jnp.float32)


def kernel(x, conv_params, bn_params, fc_params, edge_index):
    e = edge_index.shape[1]
    ept = -(-e // (NW * 1024)) * 1024
    arn = ept + NW * 16 + SEGBLK
    etot = ept * NW

    src = jnp.concatenate(
        [edge_index[0], jnp.zeros((etot - e,), jnp.int32)])
    dst = jnp.concatenate(
        [edge_index[1], jnp.full((etot - e,), NPAD, jnp.int32)])

    arena, meta = _make_bucketize(ept, arn)(src, dst)

    fc1_w, fc1_b, fc2_w, fc2_b = fc_params
    h = x
    for li, ((wl, _b, wr), (g, be)) in enumerate(zip(conv_params, bn_params)):
        d = h.shape[1]
        agg = _make_segmax(arn, d)(h, arena, meta).reshape(NPAD, d)
        hpad = h if h.shape[0] == NPAD else jnp.concatenate(
            [h, jnp.zeros((NPAD - h.shape[0], d), jnp.float32)])
        y, st = _tc_linear(agg, hpad, wl, wr)
        gb = jnp.concatenate([_row8(g, H)[0:1], _row8(be, H)[0:1],
                              jnp.zeros((6, H), jnp.float32)])
        h = _tc_bn(y, st, gb)

    out = _tc_head(h, fc1_w, _row8(fc1_b, 128), _row8(fc2_w.reshape(-1), 128),
                   _row8(jnp.broadcast_to(fc2_b, (128,)), 128))
    return out[:N, :1]


# speedup vs baseline: 1.3155x; 1.3155x over previous
import functools

import jax
import jax.numpy as jnp
from jax import lax
from jax.experimental import pallas as pl
from jax.experimental.pallas import tpu as pltpu, tpu_sc as plsc

N = 10000
H = 256
NEG_SLOPE = 0.01
EPS = 1e-5

NC, NS = 2, 16
NW = NC * NS
RANGE = 320
NPAD = NW * RANGE
NINF = float("-inf")

SEGBLK = 512
MSLOT = 16
_SC_MESH = dict(core_axis_name="c", subcore_axis_name="s")


def _wid():
    return lax.axis_index("s") * NC + lax.axis_index("c")


def _bucket(dst):
    return jnp.minimum((dst * 13108) >> 22, 31)


def _make_bucketize(ept, arn):
    nblk = ept // 1024

    @functools.partial(
        pl.kernel,
        out_type=[
            jax.ShapeDtypeStruct((NW * arn,), jnp.int32),
            jax.ShapeDtypeStruct((NW * 2 * NW * MSLOT,), jnp.int32),
        ],
        mesh=plsc.VectorSubcoreMesh(**_SC_MESH),
        scratch_types=[
            pltpu.VMEM((arn,), jnp.int32),
            pltpu.VMEM((1024,), jnp.int32),
            pltpu.VMEM((1024,), jnp.int32),
            pltpu.VMEM((16,), jnp.int32),
            pltpu.SMEM((64,), jnp.int32),
        ],
    )
    def bucketize(src_hbm, dst_hbm, arena_o, meta_o, arena, dstst, srcst, mbuf, cnt):
        w = _wid()
        iota = lax.iota(jnp.int32, 16)
        ebase = w * ept

        for b in range(NW):
            cnt[b] = 0

        def p1_blk(blk, _):
            pltpu.sync_copy(
                dst_hbm.at[pl.ds(pl.multiple_of(ebase + blk * 1024, 16), 1024)], dstst)

            def p1_chunk(ci, _):
                bv = _bucket(dstst[pl.ds(ci * 16, 16)])
                for e in range(16):
                    b = bv[e]
                    cnt[b] = cnt[b] + 1
                return 0

            lax.fori_loop(0, 64, p1_chunk, 0)
            return 0

        lax.fori_loop(0, nblk, p1_blk, 0)

        run = jnp.int32(0)
        for b in range(NW):
            c = cnt[b]
            cnt[b] = run
            cnt[32 + b] = run
            run = run + ((c + 15) & ~15)

        dummy = jnp.full((16,), RANGE * 16384, jnp.int32)

        def fill(i, _):
            arena[pl.ds(i * 16, 16)] = dummy
            return 0

        lax.fori_loop(0, arn // 16, fill, 0)

        def p2_blk(blk, _):
            base = pl.multiple_of(ebase + blk * 1024, 16)
            pltpu.sync_copy(dst_hbm.at[pl.ds(base, 1024)], dstst)
            pltpu.sync_copy(src_hbm.at[pl.ds(base, 1024)], srcst)

            def p2_chunk(ci, _):
                dv = dstst[pl.ds(ci * 16, 16)]
                sv = srcst[pl.ds(ci * 16, 16)]
                bv = _bucket(dv)
                pkv = (dv - bv * RANGE) * 16384 + sv
                for e in range(16):
                    b = bv[e]
                    pk = pkv[e]
                    p = cnt[32 + b]
                    base = p & ~15
                    old = arena[pl.ds(base, 16)]
                    arena[pl.ds(base, 16)] = jnp.where(iota == (p & 15), pk, old)
                    cnt[32 + b] = p + 1
                return 0

            lax.fori_loop(0, 64, p2_chunk, 0)
            return 0

        lax.fori_loop(0, nblk, p2_blk, 0)

        for b in range(NW):
            c16 = (cnt[32 + b] - cnt[b] + 15) & ~15
            mbuf[...] = jnp.where(iota == 0, c16, 0)
            pltpu.sync_copy(mbuf.at[pl.ds(0, MSLOT)],
                            meta_o.at[pl.ds(pl.multiple_of((b * 2) * NW * MSLOT + w * MSLOT, 16), MSLOT)])
            mbuf[...] = jnp.where(iota == 0, cnt[b], 0)
            pltpu.sync_copy(mbuf.at[pl.ds(0, MSLOT)],
                            meta_o.at[pl.ds(pl.multiple_of((b * 2 + 1) * NW * MSLOT + w * MSLOT, 16), MSLOT)])

        pltpu.sync_copy(arena.at[pl.ds(0, arn)],
                        arena_o.at[pl.ds(pl.multiple_of(w * arn, 16), arn)])

    return bucketize


def _make_segmax(arn, d):
    nvr = d // 16

    @functools.partial(
        pl.kernel,
        out_type=jax.ShapeDtypeStruct((NPAD * d,), jnp.float32),
        mesh=plsc.VectorSubcoreMesh(**_SC_MESH),
        scratch_types=[
            pltpu.VMEM(((RANGE + 1) * d,), jnp.float32),
            pltpu.VMEM((SEGBLK,), jnp.int32),
            pltpu.VMEM((16, d), jnp.float32),
            pltpu.VMEM((16, d), jnp.float32),
            pltpu.VMEM((16, d), jnp.float32),
            pltpu.VMEM((16, d), jnp.float32),
            pltpu.VMEM((2 * NW * MSLOT,), jnp.int32),
            pltpu.SemaphoreType.DMA,
            pltpu.SemaphoreType.DMA,
            pltpu.SemaphoreType.DMA,
            pltpu.SemaphoreType.DMA,
        ],
    )
    def segmax(h_hbm, arena_hbm, meta_hbm, out_hbm, acc, seg, rows0, rows1,
               rows2, rows3, meta_v, sem0, sem1, sem2, sem3):
        w = _wid()
        ninf = jnp.full((16,), NINF, jnp.float32)

        def init(i, _):
            acc[pl.ds(i * 16, 16)] = ninf
            return 0

        lax.fori_loop(0, (RANGE + 1) * d // 16, init, 0)

        pltpu.sync_copy(
            meta_hbm.at[pl.ds(pl.multiple_of(w * 2 * NW * MSLOT, 16), 2 * NW * MSLOT)],
            meta_v)

        def accum(rows, dstv):
            for e in range(16):
                base = dstv[e] * d
                for j in range(nvr):
                    a = acc[pl.ds(base + j * 16, 16)]
                    r = rows[e, pl.ds(j * 16, 16)]
                    acc[pl.ds(base + j * 16, 16)] = jnp.maximum(a, r)

        def producer(t, _):
            cntp = meta_v[pl.ds(t * MSLOT, 16)][0]
            off = meta_v[pl.ds(NW * MSLOT + t * MSLOT, 16)][0]
            abase = pl.multiple_of(t * arn + off, 16)

            def blk(bi, _):
                pltpu.sync_copy(
                    arena_hbm.at[pl.ds(pl.multiple_of(abase + bi * SEGBLK, 16), SEGBLK)],
                    seg)
                nch = jnp.minimum(cntp - bi * SEGBLK, SEGBLK) >> 4
                nquad = nch >> 2

                rbufs = (rows0, rows1, rows2, rows3)
                sems = (sem0, sem1, sem2, sem3)

                def quad(qi, _):
                    c0 = qi * 4
                    pvs = [seg[pl.ds(c0 * 16 + 16 * u, 16)] for u in range(4)]
                    cps = [
                        pltpu.async_copy(h_hbm.at[pvs[u] & 16383], rbufs[u], sems[u])
                        for u in range(4)
                    ]
                    for u in range(4):
                        cps[u].wait()
                        accum(rbufs[u], lax.shift_right_logical(pvs[u], 14))
                    return 0

                lax.fori_loop(0, nquad, quad, 0)

                def chunk(ci, _):
                    pv = seg[pl.ds(ci * 16, 16)]
                    pltpu.async_copy(h_hbm.at[pv & 16383], rows0, sem0).wait()
                    accum(rows0, lax.shift_right_logical(pv, 14))
                    return 0

                lax.fori_loop(nquad * 4, nch, chunk, 0)
                return 0

            lax.fori_loop(0, (cntp + SEGBLK - 1) >> 9, blk, 0)
            return 0

        lax.fori_loop(0, NW, producer, 0)

        def fin(i, _):
            a = acc[pl.ds(i * 16, 16)]
            acc[pl.ds(i * 16, 16)] = jnp.where(a == NINF, 0.0, a)
            return 0

        lax.fori_loop(0, RANGE * d // 16, fin, 0)
        pltpu.sync_copy(acc.at[pl.ds(0, RANGE * d)],
                        out_hbm.at[pl.ds(pl.multiple_of(w * RANGE * d, 16), RANGE * d)])

    return segmax


_GRID = NPAD // 256


def _tc_linear(agg, h, wl, wr):
    k = agg.shape[1]

    def body(agg_ref, h_ref, wl_ref, wr_ref, y_ref, st_ref, sacc):
        i = pl.program_id(0)
        y = lax.dot_general(agg_ref[...], wl_ref[...], (((1,), (1,)), ((), ())),
                            preferred_element_type=jnp.float32)
        y = y + lax.dot_general(h_ref[...], wr_ref[...], (((1,), (1,)), ((), ())),
                                preferred_element_type=jnp.float32)
        y_ref[...] = y
        rid = i * 256 + lax.broadcasted_iota(jnp.int32, (256, 1), 0)
        ym = jnp.where(rid < N, y, 0.0)

        @pl.when(i == 0)
        def _():
            sacc[...] = jnp.zeros((8, H), jnp.float32)

        sacc[0:1, :] = sacc[0:1, :] + jnp.sum(ym, axis=0, keepdims=True)
        sacc[1:2, :] = sacc[1:2, :] + jnp.sum(ym * ym, axis=0, keepdims=True)

        @pl.when(i == _GRID - 1)
        def _():
            st_ref[...] = sacc[...]

    return pl.pallas_call(
        body,
        grid=(_GRID,),
        in_specs=[
            pl.BlockSpec((256, k), lambda i: (i, 0)),
            pl.BlockSpec((256, k), lambda i: (i, 0)),
            pl.BlockSpec((H, k), lambda i: (0, 0)),
            pl.BlockSpec((H, k), lambda i: (0, 0)),
        ],
        out_specs=[
            pl.BlockSpec((256, H), lambda i: (i, 0)),
            pl.BlockSpec((8, H), lambda i: (0, 0)),
        ],
        out_shape=[
            jax.ShapeDtypeStruct((NPAD, H), jnp.float32),
            jax.ShapeDtypeStruct((8, H), jnp.float32),
        ],
        scratch_shapes=[pltpu.VMEM((8, H), jnp.float32)],
    )(agg, h, wl, wr)


def _tc_bn(y, st, gb):
    def body(y_ref, st_ref, gb_ref, o_ref):
        s = st_ref[0:1, :]
        s2 = st_ref[1:2, :]
        mean = s / N
        var = s2 / N - mean * mean
        rstd = lax.rsqrt(var + EPS)
        g = gb_ref[0:1, :]
        b = gb_ref[1:2, :]
        hn = (y_ref[...] - mean) * (rstd * g) + b
        o_ref[...] = jnp.where(hn >= 0, hn, hn * NEG_SLOPE)

    return pl.pallas_call(
        body,
        grid=(_GRID,),
        in_specs=[
            pl.BlockSpec((256, H), lambda i: (i, 0)),
            pl.BlockSpec((8, H), lambda i: (0, 0)),
            pl.BlockSpec((8, H), lambda i: (0, 0)),
        ],
        out_specs=pl.BlockSpec((256, H), lambda i: (i, 0)),
        out_shape=jax.ShapeDtypeStruct((NPAD, H), jnp.float32),
    )(y, st, gb)


def _tc_head(h, fc1w, fc1b, fc2w, fc2b):
    def body(h_ref, w1_ref, b1_ref, w2_ref, b2_ref, o_ref):
        h2 = lax.dot_general(h_ref[...], w1_ref[...], (((1,), (1,)), ((), ())),
                             preferred_element_type=jnp.float32)
        h2 = h2 + b1_ref[0:1, :]
        h2 = jnp.where(h2 >= 0, h2, h2 * NEG_SLOPE)
        o = jnp.sum(h2 * w2_ref[0:1, :], axis=1, keepdims=True) + b2_ref[0, 0]
        o_ref[...] = jnp.broadcast_to(o, (256, 128))

    return pl.pallas_call(
        body,
        grid=(_GRID,),
        in_specs=[
            pl.BlockSpec((256, H), lambda i: (i, 0)),
            pl.BlockSpec((128, H), lambda i: (0, 0)),
            pl.BlockSpec((8, 128), lambda i: (0, 0)),
            pl.BlockSpec((8, 128), lambda i: (0, 0)),
            pl.BlockSpec((8, 128), lambda i: (0, 0)),
        ],
        out_specs=pl.BlockSpec((256, 128), lambda i: (i, 0)),
        out_shape=jax.ShapeDtypeStruct((NPAD, 128), jnp.float32),
    )(h, fc1w, fc1b, fc2w, fc2b)


def _row8(v, width):
    return jnp.broadcast_to(v.reshape(1, -1), (8, width)).astype(jnp.float32)


def kernel(x, conv_params, bn_params, fc_params, edge_index):
    e = edge_index.shape[1]
    ept = -(-e // (NW * 1024)) * 1024
    arn = ept + NW * 16 + SEGBLK
    etot = ept * NW

    src = jnp.concatenate(
        [edge_index[0], jnp.zeros((etot - e,), jnp.int32)])
    dst = jnp.concatenate(
        [edge_index[1], jnp.full((etot - e,), NPAD, jnp.int32)])

    arena, meta = _make_bucketize(ept, arn)(src, dst)

    fc1_w, fc1_b, fc2_w, fc2_b = fc_params
    h = x
    for li, ((wl, _b, wr), (g, be)) in enumerate(zip(conv_params, bn_params)):
        d = h.shape[1]
        agg = _make_segmax(arn, d)(h, arena, meta).reshape(NPAD, d)
        hpad = h if h.shape[0] == NPAD else jnp.concatenate(
            [h, jnp.zeros((NPAD - h.shape[0], d), jnp.float32)])
        y, st = _tc_linear(agg, hpad, wl, wr)
        gb = jnp.concatenate([_row8(g, H)[0:1], _row8(be, H)[0:1],
                              jnp.zeros((6, H), jnp.float32)])
        h = _tc_bn(y, st, gb)

    out = _tc_head(h, fc1_w, _row8(fc1_b, 128), _row8(fc2_w.reshape(-1), 128),
                   _row8(jnp.broadcast_to(fc2_b, (128,)), 128))
    return out[:N, :1]
